# baseline (speedup 1.0000x reference)
import functools

import jax
import jax.numpy as jnp
from jax import lax
from jax.experimental import pallas as pl
from jax.experimental.pallas import tpu as pltpu
from jax.experimental.pallas import tpu_sc as plsc

VOCAB = 1000000
EMBED_DIM = 32
BATCH = 16384
N_FIELDS = 26
PACK = 128 // EMBED_DIM

NW = 32
B_FLAT = BATCH * N_FIELDS
PER_W = B_FLAT // NW
BLK = 1024
NBLK = PER_W // BLK
SUB = 256
NSUB = BLK // SUB


def _body(idx_hbm, emb_hbm, w_hbm, out_hbm, idxb, tid, e_v, w_v, o_v, sem):
    wid = lax.axis_index("s") * 2 + lax.axis_index("c")

    def blk_body(b, carry):
        pltpu.sync_copy(idx_hbm.at[pl.ds((wid * NBLK + b) * 8, 8)], idxb)
        for r in range(8):
            for v in range(8):
                sl = pl.ds(v * 16, 16)
                tid[r, sl] = lax.shift_right_logical(idxb[r, sl], PACK // 2)

        def sub_body(sub, scarry):
            cps = []
            for j in range(SUB // 128):
                dst = pl.ds(j * 128, 128)
                row = sub * (SUB // 128) + j
                cps.append(pltpu.async_copy(emb_hbm.at[tid.at[row]], e_v.at[dst], sem))
                cps.append(pltpu.async_copy(w_hbm.at[tid.at[row]], w_v.at[dst], sem))
            for cp in cps:
                cp.wait()

            def mul_body(g, mcarry):
                rb = sub * SUB + g * 16
                row = lax.shift_right_logical(rb, 7)
                col = lax.rem(rb, 128)
                ivec = idxb[row, pl.ds(col, 16)]
                offvec = lax.rem(ivec, PACK) * EMBED_DIM
                for k in range(16):
                    off = offvec[k]
                    r = g * 16 + k
                    for h in range(2):
                        src = pl.ds(off + h * 16, 16)
                        dst = pl.ds((k % 4) * 32 + h * 16, 16)
                        o_v[g * 4 + k // 4, dst] = e_v[r, src] * w_v[r, src]
                return mcarry

            lax.fori_loop(0, SUB // 16, mul_body, 0)
            obase = wid * (PER_W // PACK) + (b * NSUB + sub) * (SUB // PACK)
            pltpu.sync_copy(o_v, out_hbm.at[pl.ds(obase, SUB // PACK)])
            return scarry

        lax.fori_loop(0, NSUB, sub_body, 0)
        return carry

    lax.fori_loop(0, NBLK, blk_body, 0)


_lookup = functools.partial(
    pl.kernel,
    out_type=jax.ShapeDtypeStruct((B_FLAT // PACK, 128), jnp.float32),
    mesh=plsc.VectorSubcoreMesh(core_axis_name="c", subcore_axis_name="s"),
    scratch_types=[
        pltpu.VMEM((8, 128), jnp.int32),
        pltpu.VMEM((8, 128), jnp.int32),
        pltpu.VMEM((SUB, 128), jnp.float32),
        pltpu.VMEM((SUB, 128), jnp.float32),
        pltpu.VMEM((SUB // PACK, 128), jnp.float32),
        pltpu.SemaphoreType.DMA,
    ],
)(_body)


@jax.jit
def kernel(inputs, embeddings, w):
    idx = inputs.astype(jnp.int32).reshape(B_FLAT // 128, 128)
    emb2 = embeddings.reshape(VOCAB // PACK, 128)
    w2 = w.reshape(VOCAB // PACK, 128)
    out = _lookup(idx, emb2, w2)
    return out.reshape(BATCH, N_FIELDS, EMBED_DIM)


# speedup vs baseline: 1.7470x; 1.7470x over previous
import functools

import jax
import jax.numpy as jnp
from jax import lax
from jax.experimental import pallas as pl
from jax.experimental.pallas import tpu as pltpu
from jax.experimental.pallas import tpu_sc as plsc

VOCAB = 1000000
EMBED_DIM = 32
BATCH = 16384
N_FIELDS = 26
PACK = 128 // EMBED_DIM

NW = 32
VBLK = 512
NVBLK = VOCAB // VBLK
V_TAIL = VOCAB - NVBLK * VBLK
P_LEN = VOCAB * EMBED_DIM

BBLK = 512
B_FLAT = BATCH * N_FIELDS
N_UNITS = B_FLAT // BBLK
UNITS_PER_W = N_UNITS // NW
OUT_LEN = B_FLAT * EMBED_DIM

_GATHER_DN = lax.GatherDimensionNumbers(
    offset_dims=(), collapsed_slice_dims=(0,), start_index_map=(0,))


def _lane_perm(v, perm2d):
    return lax.gather(v, perm2d, _GATHER_DN, (1,),
                      mode=lax.GatherScatterMode.PROMISE_IN_BOUNDS)


def _xpose16(rows):
    lanes = lax.iota(jnp.int32, 16)
    for d in (8, 4, 2, 1):
        mask = (lanes & d) == 0
        perm2d = (lanes ^ d)[:, None]
        nxt = list(rows)
        for r in range(16):
            if r & d:
                continue
            a, b = rows[r], rows[r + d]
            nxt[r] = jnp.where(mask, a, _lane_perm(b, perm2d))
            nxt[r + d] = jnp.where(mask, _lane_perm(a, perm2d), b)
        rows = nxt
    return rows


def _phase1(embT, wT, tail_p, p_out, eb, wb, pb, tb, sem):
    wid = lax.axis_index("s") * 2 + lax.axis_index("c")
    nk = NVBLK // NW + 1

    def blk_body(k, carry):
        blk = wid + k * NW

        @pl.when(blk < NVBLK)
        def _():
            v0 = blk * VBLK
            cp_e = pltpu.async_copy(embT.at[:, pl.ds(v0, VBLK)], eb, sem)
            cp_w = pltpu.async_copy(wT.at[:, pl.ds(v0, VBLK)], wb, sem)
            cp_e.wait()
            cp_w.wait()

            def vb_body(j, c2):
                for ch in range(2):
                    sl = pl.ds(j * 16, 16)
                    rows = [eb[ch * 16 + i, sl] * wb[ch * 16 + i, sl]
                            for i in range(16)]
                    t = _xpose16(rows)
                    for i in range(16):
                        pb[pl.ds((j * 16 + i) * EMBED_DIM + ch * 16, 16)] = t[i]
                return c2

            lax.fori_loop(0, VBLK // 16, vb_body, 0)
            pltpu.sync_copy(pb, p_out.at[pl.ds(v0 * EMBED_DIM, VBLK * EMBED_DIM)])

        return carry

    lax.fori_loop(0, nk, blk_body, 0)

    @pl.when(wid == NW - 1)
    def _():
        pltpu.sync_copy(tail_p, tb)
        for r in range(V_TAIL * EMBED_DIM // 128):
            for h in range(8):
                pb[pl.ds(r * 128 + h * 16, 16)] = tb[r, pl.ds(h * 16, 16)]
        pltpu.sync_copy(
            pb.at[pl.ds(0, V_TAIL * EMBED_DIM)],
            p_out.at[pl.ds(NVBLK * VBLK * EMBED_DIM, V_TAIL * EMBED_DIM)])


def _phase2(idx_flat, p_in, out, ib, tid, offb, gb, sb, sem):
    wid = lax.axis_index("s") * 2 + lax.axis_index("c")

    def unit_body(k, carry):
        u = wid + k * NW
        base = u * BBLK
        pltpu.sync_copy(idx_flat.at[pl.ds(base, BBLK)], ib)
        for h in range(BBLK // 16):
            sl = pl.ds(h * 16, 16)
            iv = ib[sl]
            tid[h // 8, pl.ds((h % 8) * 16, 16)] = lax.shift_right_logical(iv, 2)
            offb[sl] = lax.rem(iv, PACK) * EMBED_DIM
        cps = [
            pltpu.async_copy(p_in.at[tid.at[j]], gb.at[pl.ds(j * 128, 128)], sem)
            for j in range(BBLK // 128)
        ]
        for cp in cps:
            cp.wait()

        def g_body(g, c2):
            offv = offb[pl.ds(g * 16, 16)]
            for kk in range(16):
                off_k = offv[kk]
                b = g * 16 + kk
                for h2 in range(2):
                    sb[pl.ds(b * EMBED_DIM + h2 * 16, 16)] = (
                        gb[b, pl.ds(off_k + h2 * 16, 16)])
            return c2

        lax.fori_loop(0, BBLK // 16, g_body, 0)
        pltpu.sync_copy(sb, out.at[pl.ds(base * EMBED_DIM, BBLK * EMBED_DIM)])
        return carry

    lax.fori_loop(0, UNITS_PER_W, unit_body, 0)


_MESH = plsc.VectorSubcoreMesh(core_axis_name="c", subcore_axis_name="s")

_p1 = functools.partial(
    pl.kernel,
    out_type=jax.ShapeDtypeStruct((P_LEN,), jnp.float32),
    mesh=_MESH,
    scratch_types=[
        pltpu.VMEM((EMBED_DIM, VBLK), jnp.float32),
        pltpu.VMEM((EMBED_DIM, VBLK), jnp.float32),
        pltpu.VMEM((VBLK * EMBED_DIM,), jnp.float32),
        pltpu.VMEM((V_TAIL * EMBED_DIM // 128, 128), jnp.float32),
        pltpu.SemaphoreType.DMA,
    ],
)(_phase1)

_p2 = functools.partial(
    pl.kernel,
    out_type=jax.ShapeDtypeStruct((OUT_LEN,), jnp.float32),
    mesh=_MESH,
    scratch_types=[
        pltpu.VMEM((BBLK,), jnp.int32),
        pltpu.VMEM((BBLK // 128, 128), jnp.int32),
        pltpu.VMEM((BBLK,), jnp.int32),
        pltpu.VMEM((BBLK, 128), jnp.float32),
        pltpu.VMEM((BBLK * EMBED_DIM,), jnp.float32),
        pltpu.SemaphoreType.DMA,
    ],
)(_phase2)


@jax.jit
def kernel(inputs, embeddings, w):
    embT = embeddings.T
    wT = w.T
    tail_v0 = NVBLK * VBLK
    tail_p = (embeddings[tail_v0:] * w[tail_v0:]).reshape(
        V_TAIL * EMBED_DIM // 128, 128)
    idx_flat = inputs.reshape(-1).astype(jnp.int32)
    p = _p1(embT, wT, tail_p)
    p2d = p.reshape(P_LEN // 128, 128)
    out = _p2(idx_flat, p2d)
    return out.reshape(BATCH, N_FIELDS, EMBED_DIM)


# speedup vs baseline: 2.2289x; 1.2758x over previous
import functools

import jax
import jax.numpy as jnp
from jax import lax
from jax.experimental import pallas as pl
from jax.experimental.pallas import tpu as pltpu
from jax.experimental.pallas import tpu_sc as plsc

VOCAB = 1000000
EMBED_DIM = 32
BATCH = 16384
N_FIELDS = 26
PACK = 128 // EMBED_DIM

NW = 32
VBLK = 512
PSZ = VBLK * EMBED_DIM
NPIPE = 60
NEXTRA = VOCAB // VBLK - NPIPE * NW
V_TAIL = VOCAB - (VOCAB // VBLK) * VBLK
P_LEN = VOCAB * EMBED_DIM

BBLK = 256
B_FLAT = BATCH * N_FIELDS
N_UNITS = B_FLAT // BBLK
UPW = N_UNITS // NW
OSZ = BBLK * EMBED_DIM
OUT_LEN = B_FLAT * EMBED_DIM

_GATHER_DN = lax.GatherDimensionNumbers(
    offset_dims=(), collapsed_slice_dims=(0,), start_index_map=(0,))


def _lane_perm(v, perm2d):
    return lax.gather(v, perm2d, _GATHER_DN, (1,),
                      mode=lax.GatherScatterMode.PROMISE_IN_BOUNDS)


def _xpose16(rows):
    lanes = lax.iota(jnp.int32, 16)
    for d in (8, 4, 2, 1):
        mask = (lanes & d) == 0
        perm2d = (lanes ^ d)[:, None]
        nxt = list(rows)
        for r in range(16):
            if r & d:
                continue
            a, b = rows[r], rows[r + d]
            nxt[r] = jnp.where(mask, a, _lane_perm(b, perm2d))
            nxt[r + d] = jnp.where(mask, _lane_perm(a, perm2d), b)
        rows = nxt
    return rows


def _phase1(embT, wT, tail_p, p_out, eb, wb, pb, tb,
            sl0, sl1, so0, so1):
    wid = lax.axis_index("s") * 2 + lax.axis_index("c")
    sl = (sl0, sl1)
    so = (so0, so1)

    def issue_loads(panel, par):
        v0 = panel * VBLK
        pltpu.async_copy(embT.at[:, pl.ds(v0, VBLK)], eb.at[par], sl[par])
        pltpu.async_copy(wT.at[:, pl.ds(v0, VBLK)], wb.at[par], sl[par])

    def wait_loads(panel, par):
        v0 = panel * VBLK
        pltpu.make_async_copy(embT.at[:, pl.ds(v0, VBLK)], eb.at[par], sl[par]).wait()
        pltpu.make_async_copy(wT.at[:, pl.ds(v0, VBLK)], wb.at[par], sl[par]).wait()

    def compute_panel(par, nrows=VBLK // 16):
        def vb_body(j, c2):
            for ch in range(2):
                s16 = pl.ds(j * 16, 16)
                rows = [eb[par, ch * 16 + i, s16] * wb[par, ch * 16 + i, s16]
                        for i in range(16)]
                t = _xpose16(rows)
                for i in range(16):
                    pb[par, pl.ds((j * 16 + i) * EMBED_DIM + ch * 16, 16)] = t[i]
            return c2

        lax.fori_loop(0, nrows, vb_body, 0)

    for par in range(2):
        issue_loads(wid + par * NW, par)

    def blk_body(k2, carry):
        for par in range(2):
            k = k2 * 2 + par
            panel = wid + k * NW
            wait_loads(panel, par)

            @pl.when(k2 >= 1)
            def _():
                pltpu.make_async_copy(
                    pb.at[par], p_out.at[pl.ds(0, PSZ)], so[par]).wait()

            compute_panel(par)
            pltpu.async_copy(
                pb.at[par], p_out.at[pl.ds(panel * PSZ, PSZ)], so[par])

            @pl.when(k2 < NPIPE // 2 - 1)
            def _():
                issue_loads(wid + (k + 2) * NW, par)

        return carry

    lax.fori_loop(0, NPIPE // 2, blk_body, 0)
    for par in range(2):
        pltpu.make_async_copy(pb.at[par], p_out.at[pl.ds(0, PSZ)], so[par]).wait()

    extra = NPIPE * NW + wid
    pltpu.sync_copy(embT.at[:, pl.ds(extra * VBLK, VBLK)], eb.at[0])
    pltpu.sync_copy(wT.at[:, pl.ds(extra * VBLK, VBLK)], wb.at[0])
    compute_panel(0)
    pltpu.sync_copy(pb.at[0], p_out.at[pl.ds(extra * PSZ, PSZ)])

    @pl.when(wid == NW - 1)
    def _():
        last = NPIPE * NW + NW
        pltpu.sync_copy(embT.at[:, pl.ds(last * VBLK, VBLK)], eb.at[1])
        pltpu.sync_copy(wT.at[:, pl.ds(last * VBLK, VBLK)], wb.at[1])
        compute_panel(1)
        pltpu.sync_copy(pb.at[1], p_out.at[pl.ds(last * PSZ, PSZ)])

        pltpu.sync_copy(tail_p, tb)
        for r in range(V_TAIL * EMBED_DIM // 128):
            for h in range(8):
                pb[0, pl.ds(r * 128 + h * 16, 16)] = tb[r, pl.ds(h * 16, 16)]
        pltpu.sync_copy(
            pb.at[0, pl.ds(0, V_TAIL * EMBED_DIM)],
            p_out.at[pl.ds((last + 1) * PSZ, V_TAIL * EMBED_DIM)])


def _phase2(idx_flat, p_in, out, ib, tid, offb, gb, sb,
            sg0, sg1, so0, so1):
    wid = lax.axis_index("s") * 2 + lax.axis_index("c")
    sg = (sg0, sg1)
    so = (so0, so1)

    def prep(k, par):
        base = (wid + k * NW) * BBLK
        pltpu.sync_copy(idx_flat.at[pl.ds(base, BBLK)], ib.at[par])
        for h in range(BBLK // 16):
            s16 = pl.ds(h * 16, 16)
            iv = ib[par, s16]
            tid[par, h // 8, pl.ds((h % 8) * 16, 16)] = (
                lax.shift_right_logical(iv, 2))
            offb[par, s16] = lax.rem(iv, PACK) * EMBED_DIM
        for j in range(BBLK // 128):
            pltpu.async_copy(
                p_in.at[tid.at[par, j]],
                gb.at[par, pl.ds(j * 128, 128)], sg[par])

    def wait_gathers(par):
        for j in range(BBLK // 128):
            pltpu.make_async_copy(
                p_in.at[tid.at[par, j]],
                gb.at[par, pl.ds(j * 128, 128)], sg[par]).wait()

    for par in range(2):
        prep(par, par)

    def unit_body(k2, carry):
        for par in range(2):
            k = k2 * 2 + par
            base = (wid + k * NW) * BBLK
            wait_gathers(par)

            @pl.when(k2 >= 1)
            def _():
                pltpu.make_async_copy(
                    sb.at[par], out.at[pl.ds(0, OSZ)], so[par]).wait()

            def g_body(g, c2):
                offv = offb[par, pl.ds(g * 16, 16)]
                for kk in range(16):
                    off_k = offv[kk]
                    b = g * 16 + kk
                    for h2 in range(2):
                        sb[par, pl.ds(b * EMBED_DIM + h2 * 16, 16)] = (
                            gb[par, b, pl.ds(off_k + h2 * 16, 16)])
                return c2

            lax.fori_loop(0, BBLK // 16, g_body, 0)
            pltpu.async_copy(
                sb.at[par], out.at[pl.ds(base * EMBED_DIM, OSZ)], so[par])

            @pl.when(k2 < UPW // 2 - 1)
            def _():
                prep(k + 2, par)

        return carry

    lax.fori_loop(0, UPW // 2, unit_body, 0)
    for par in range(2):
        pltpu.make_async_copy(sb.at[par], out.at[pl.ds(0, OSZ)], so[par]).wait()


_MESH = plsc.VectorSubcoreMesh(core_axis_name="c", subcore_axis_name="s")

_p1 = functools.partial(
    pl.kernel,
    out_type=jax.ShapeDtypeStruct((P_LEN,), jnp.float32),
    mesh=_MESH,
    scratch_types=[
        pltpu.VMEM((2, EMBED_DIM, VBLK), jnp.float32),
        pltpu.VMEM((2, EMBED_DIM, VBLK), jnp.float32),
        pltpu.VMEM((2, PSZ), jnp.float32),
        pltpu.VMEM((V_TAIL * EMBED_DIM // 128, 128), jnp.float32),
        pltpu.SemaphoreType.DMA,
        pltpu.SemaphoreType.DMA,
        pltpu.SemaphoreType.DMA,
        pltpu.SemaphoreType.DMA,
    ],
)(_phase1)

_p2 = functools.partial(
    pl.kernel,
    out_type=jax.ShapeDtypeStruct((OUT_LEN,), jnp.float32),
    mesh=_MESH,
    scratch_types=[
        pltpu.VMEM((2, BBLK), jnp.int32),
        pltpu.VMEM((2, BBLK // 128, 128), jnp.int32),
        pltpu.VMEM((2, BBLK), jnp.int32),
        pltpu.VMEM((2, BBLK, 128), jnp.float32),
        pltpu.VMEM((2, OSZ), jnp.float32),
        pltpu.SemaphoreType.DMA,
        pltpu.SemaphoreType.DMA,
        pltpu.SemaphoreType.DMA,
        pltpu.SemaphoreType.DMA,
    ],
)(_phase2)


@jax.jit
def kernel(inputs, embeddings, w):
    embT = embeddings.T
    wT = w.T
    tail_v0 = (VOCAB // VBLK) * VBLK
    tail_p = (embeddings[tail_v0:] * w[tail_v0:]).reshape(
        V_TAIL * EMBED_DIM // 128, 128)
    idx_flat = inputs.reshape(-1).astype(jnp.int32)
    p = _p1(embT, wT, tail_p)
    p2d = p.reshape(P_LEN // 128, 128)
    out = _p2(idx_flat, p2d)
    return out.reshape(BATCH, N_FIELDS, EMBED_DIM)


# speedup vs baseline: 3.8827x; 1.7420x over previous
import functools

import jax
import jax.numpy as jnp
from jax import lax
from jax.experimental import pallas as pl
from jax.experimental.pallas import tpu as pltpu
from jax.experimental.pallas import tpu_sc as plsc

VOCAB = 1000000
EMBED_DIM = 32
BATCH = 16384
N_FIELDS = 26
PACK = 128 // EMBED_DIM

NW = 32
VBLK = 512
PSZ = VBLK * EMBED_DIM
NPIPE = 60
NEXTRA = VOCAB // VBLK - NPIPE * NW
V_TAIL = VOCAB - (VOCAB // VBLK) * VBLK
P_LEN = VOCAB * EMBED_DIM

BBLK = 256
B_FLAT = BATCH * N_FIELDS
N_UNITS = B_FLAT // BBLK
UPW = N_UNITS // NW
OSZ = BBLK * EMBED_DIM
OUT_LEN = B_FLAT * EMBED_DIM

_GATHER_DN = lax.GatherDimensionNumbers(
    offset_dims=(), collapsed_slice_dims=(0,), start_index_map=(0,))


def _lane_perm(v, perm2d):
    return lax.gather(v, perm2d, _GATHER_DN, (1,),
                      mode=lax.GatherScatterMode.PROMISE_IN_BOUNDS)


def _xpose16(rows):
    lanes = lax.iota(jnp.int32, 16)
    for d in (8, 4, 2, 1):
        mask = (lanes & d) == 0
        perm2d = (lanes ^ d)[:, None]
        nxt = list(rows)
        for r in range(16):
            if r & d:
                continue
            a, b = rows[r], rows[r + d]
            nxt[r] = jnp.where(mask, a, _lane_perm(b, perm2d))
            nxt[r + d] = jnp.where(mask, _lane_perm(a, perm2d), b)
        rows = nxt
    return rows


def _phase1(embT, wT, tail_p, p_out, eb, wb, pb, tb,
            sl0, sl1, so0, so1):
    wid = lax.axis_index("s") * 2 + lax.axis_index("c")
    sl = (sl0, sl1)
    so = (so0, so1)

    def issue_loads(panel, par):
        v0 = panel * VBLK
        pltpu.async_copy(embT.at[:, pl.ds(v0, VBLK)], eb.at[par], sl[par])
        pltpu.async_copy(wT.at[:, pl.ds(v0, VBLK)], wb.at[par], sl[par])

    def wait_loads(panel, par):
        v0 = panel * VBLK
        pltpu.make_async_copy(embT.at[:, pl.ds(v0, VBLK)], eb.at[par], sl[par]).wait()
        pltpu.make_async_copy(wT.at[:, pl.ds(v0, VBLK)], wb.at[par], sl[par]).wait()

    def compute_panel(par, nrows=VBLK // 16):
        def vb_body(j, c2):
            for ch in range(2):
                s16 = pl.ds(j * 16, 16)
                rows = [eb[par, ch * 16 + i, s16] * wb[par, ch * 16 + i, s16]
                        for i in range(16)]
                t = _xpose16(rows)
                for i in range(16):
                    pb[par, pl.ds((j * 16 + i) * EMBED_DIM + ch * 16, 16)] = t[i]
            return c2

        lax.fori_loop(0, nrows, vb_body, 0)

    for par in range(2):
        issue_loads(wid + par * NW, par)

    def blk_body(k2, carry):
        for par in range(2):
            k = k2 * 2 + par
            panel = wid + k * NW
            wait_loads(panel, par)

            @pl.when(k2 >= 1)
            def _():
                pltpu.make_async_copy(
                    pb.at[par], p_out.at[pl.ds(0, PSZ)], so[par]).wait()

            compute_panel(par)
            pltpu.async_copy(
                pb.at[par], p_out.at[pl.ds(panel * PSZ, PSZ)], so[par])

            @pl.when(k2 < NPIPE // 2 - 1)
            def _():
                issue_loads(wid + (k + 2) * NW, par)

        return carry

    lax.fori_loop(0, NPIPE // 2, blk_body, 0)
    for par in range(2):
        pltpu.make_async_copy(pb.at[par], p_out.at[pl.ds(0, PSZ)], so[par]).wait()

    extra = NPIPE * NW + wid
    pltpu.sync_copy(embT.at[:, pl.ds(extra * VBLK, VBLK)], eb.at[0])
    pltpu.sync_copy(wT.at[:, pl.ds(extra * VBLK, VBLK)], wb.at[0])
    compute_panel(0)
    pltpu.sync_copy(pb.at[0], p_out.at[pl.ds(extra * PSZ, PSZ)])

    @pl.when(wid == NW - 1)
    def _():
        last = NPIPE * NW + NW
        pltpu.sync_copy(embT.at[:, pl.ds(last * VBLK, VBLK)], eb.at[1])
        pltpu.sync_copy(wT.at[:, pl.ds(last * VBLK, VBLK)], wb.at[1])
        compute_panel(1)
        pltpu.sync_copy(pb.at[1], p_out.at[pl.ds(last * PSZ, PSZ)])

        pltpu.sync_copy(tail_p, tb)
        for r in range(V_TAIL * EMBED_DIM // 128):
            for h in range(8):
                pb[0, pl.ds(r * 128 + h * 16, 16)] = tb[r, pl.ds(h * 16, 16)]
        pltpu.sync_copy(
            pb.at[0, pl.ds(0, V_TAIL * EMBED_DIM)],
            p_out.at[pl.ds((last + 1) * PSZ, V_TAIL * EMBED_DIM)])


def _phase2(idxT, p_in, out, ib, tid, offb, gb, ob,
            sg0, sg1, so0, so1):
    wid = lax.axis_index("s") * 2 + lax.axis_index("c")
    sg = (sg0, sg1)
    so = (so0, so1)
    bc_per_f = BATCH // BBLK

    def unit_fb(k):
        u = wid + k * NW
        return u // bc_per_f, lax.rem(u, bc_per_f) * BBLK

    def prep(k, par):
        f, b0 = unit_fb(k)
        pltpu.sync_copy(idxT.at[pl.ds(f, 1), pl.ds(b0, BBLK)], ib.at[par])
        for h in range(BBLK // 16):
            s16 = pl.ds(h * 16, 16)
            iv = ib[par, 0, s16]
            tid[par, h // 8, pl.ds((h % 8) * 16, 16)] = (
                lax.shift_right_logical(iv, 2))
            offb[par, s16] = lax.rem(iv, PACK) * EMBED_DIM
        for j in range(BBLK // 128):
            pltpu.async_copy(
                p_in.at[tid.at[par, j]],
                gb.at[par, pl.ds(j * 128, 128)], sg[par])

    def wait_gathers(par):
        for j in range(BBLK // 128):
            pltpu.make_async_copy(
                p_in.at[tid.at[par, j]],
                gb.at[par, pl.ds(j * 128, 128)], sg[par]).wait()

    for par in range(2):
        prep(par, par)

    def unit_body(k2, carry):
        for par in range(2):
            k = k2 * 2 + par
            f, b0 = unit_fb(k)
            wait_gathers(par)

            @pl.when(k2 >= 1)
            def _():
                pltpu.make_async_copy(
                    ob.at[par], out.at[0, :, pl.ds(0, BBLK)], so[par]).wait()

            def g_body(g, c2):
                offv = offb[par, pl.ds(g * 16, 16)]
                offs = [offv[kk] for kk in range(16)]
                for ch in range(2):
                    rows = [
                        gb[par, g * 16 + kk, pl.ds(offs[kk] + ch * 16, 16)]
                        for kk in range(16)
                    ]
                    t = _xpose16(rows)
                    for i in range(16):
                        ob[par, ch * 16 + i, pl.ds(g * 16, 16)] = t[i]
                return c2

            lax.fori_loop(0, BBLK // 16, g_body, 0)
            pltpu.async_copy(
                ob.at[par], out.at[f, :, pl.ds(b0, BBLK)], so[par])

            @pl.when(k2 < UPW // 2 - 1)
            def _():
                prep(k + 2, par)

        return carry

    lax.fori_loop(0, UPW // 2, unit_body, 0)
    for par in range(2):
        pltpu.make_async_copy(
            ob.at[par], out.at[0, :, pl.ds(0, BBLK)], so[par]).wait()


_MESH = plsc.VectorSubcoreMesh(core_axis_name="c", subcore_axis_name="s")

_p1 = functools.partial(
    pl.kernel,
    out_type=jax.ShapeDtypeStruct((P_LEN,), jnp.float32),
    mesh=_MESH,
    scratch_types=[
        pltpu.VMEM((2, EMBED_DIM, VBLK), jnp.float32),
        pltpu.VMEM((2, EMBED_DIM, VBLK), jnp.float32),
        pltpu.VMEM((2, PSZ), jnp.float32),
        pltpu.VMEM((V_TAIL * EMBED_DIM // 128, 128), jnp.float32),
        pltpu.SemaphoreType.DMA,
        pltpu.SemaphoreType.DMA,
        pltpu.SemaphoreType.DMA,
        pltpu.SemaphoreType.DMA,
    ],
)(_phase1)

_p2 = functools.partial(
    pl.kernel,
    out_type=jax.ShapeDtypeStruct((N_FIELDS, EMBED_DIM, BATCH), jnp.float32),
    mesh=_MESH,
    scratch_types=[
        pltpu.VMEM((2, 1, BBLK), jnp.int32),
        pltpu.VMEM((2, BBLK // 128, 128), jnp.int32),
        pltpu.VMEM((2, BBLK), jnp.int32),
        pltpu.VMEM((2, BBLK, 128), jnp.float32),
        pltpu.VMEM((2, EMBED_DIM, BBLK), jnp.float32),
        pltpu.SemaphoreType.DMA,
        pltpu.SemaphoreType.DMA,
        pltpu.SemaphoreType.DMA,
        pltpu.SemaphoreType.DMA,
    ],
)(_phase2)


@jax.jit
def kernel(inputs, embeddings, w):
    embT = embeddings.T
    wT = w.T
    tail_v0 = (VOCAB // VBLK) * VBLK
    tail_p = (embeddings[tail_v0:] * w[tail_v0:]).reshape(
        V_TAIL * EMBED_DIM // 128, 128)
    idxT = inputs.astype(jnp.int32).T
    p = _p1(embT, wT, tail_p)
    p2d = p.reshape(P_LEN // 128, 128)
    outT = _p2(idxT, p2d)
    return outT.transpose(2, 0, 1)


# speedup vs baseline: 3.9421x; 1.0153x over previous
import functools

import jax
import jax.numpy as jnp
from jax import lax
from jax.experimental import pallas as pl
from jax.experimental.pallas import tpu as pltpu
from jax.experimental.pallas import tpu_sc as plsc

VOCAB = 1000000
EMBED_DIM = 32
BATCH = 16384
N_FIELDS = 26
PACK = 128 // EMBED_DIM

NW = 32
VBLK = 512
PSZ = VBLK * EMBED_DIM
NPIPE = 60
NEXTRA = VOCAB // VBLK - NPIPE * NW
V_TAIL = VOCAB - (VOCAB // VBLK) * VBLK
P_LEN = VOCAB * EMBED_DIM

BBLK = 256
B_FLAT = BATCH * N_FIELDS
N_UNITS = B_FLAT // BBLK
UPW = N_UNITS // NW
OSZ = BBLK * EMBED_DIM
OUT_LEN = B_FLAT * EMBED_DIM

_GATHER_DN = lax.GatherDimensionNumbers(
    offset_dims=(), collapsed_slice_dims=(0,), start_index_map=(0,))


def _lane_perm(v, perm2d):
    return lax.gather(v, perm2d, _GATHER_DN, (1,),
                      mode=lax.GatherScatterMode.PROMISE_IN_BOUNDS)


def _xpose16(rows):
    lanes = lax.iota(jnp.int32, 16)
    for d in (8, 4, 2, 1):
        mask = (lanes & d) == 0
        perm2d = (lanes ^ d)[:, None]
        nxt = list(rows)
        for r in range(16):
            if r & d:
                continue
            a, b = rows[r], rows[r + d]
            nxt[r] = jnp.where(mask, a, _lane_perm(b, perm2d))
            nxt[r + d] = jnp.where(mask, _lane_perm(a, perm2d), b)
        rows = nxt
    return rows


def _phase1(embT, wT, tail_p, p_out, eb, wb, pb, tb,
            sl0, sl1, so0, so1):
    wid = lax.axis_index("s") * 2 + lax.axis_index("c")
    sl = (sl0, sl1)
    so = (so0, so1)

    def issue_loads(panel, par):
        v0 = panel * VBLK
        pltpu.async_copy(embT.at[:, pl.ds(v0, VBLK)], eb.at[par], sl[par])
        pltpu.async_copy(wT.at[:, pl.ds(v0, VBLK)], wb.at[par], sl[par])

    def wait_loads(panel, par):
        v0 = panel * VBLK
        pltpu.make_async_copy(embT.at[:, pl.ds(v0, VBLK)], eb.at[par], sl[par]).wait()
        pltpu.make_async_copy(wT.at[:, pl.ds(v0, VBLK)], wb.at[par], sl[par]).wait()

    def compute_panel(par, nrows=VBLK // 16):
        def vb_body(j, c2):
            for ch in range(2):
                s16 = pl.ds(j * 16, 16)
                rows = [eb[par, ch * 16 + i, s16] * wb[par, ch * 16 + i, s16]
                        for i in range(16)]
                t = _xpose16(rows)
                for i in range(16):
                    pb[par, pl.ds((j * 16 + i) * EMBED_DIM + ch * 16, 16)] = t[i]
            return c2

        lax.fori_loop(0, nrows, vb_body, 0, unroll=2)

    for par in range(2):
        issue_loads(wid + par * NW, par)

    def blk_body(k2, carry):
        for par in range(2):
            k = k2 * 2 + par
            panel = wid + k * NW
            wait_loads(panel, par)

            @pl.when(k2 >= 1)
            def _():
                pltpu.make_async_copy(
                    pb.at[par], p_out.at[pl.ds(0, PSZ)], so[par]).wait()

            compute_panel(par)
            pltpu.async_copy(
                pb.at[par], p_out.at[pl.ds(panel * PSZ, PSZ)], so[par])

            @pl.when(k2 < NPIPE // 2 - 1)
            def _():
                issue_loads(wid + (k + 2) * NW, par)

        return carry

    lax.fori_loop(0, NPIPE // 2, blk_body, 0)
    for par in range(2):
        pltpu.make_async_copy(pb.at[par], p_out.at[pl.ds(0, PSZ)], so[par]).wait()

    extra = NPIPE * NW + wid
    pltpu.sync_copy(embT.at[:, pl.ds(extra * VBLK, VBLK)], eb.at[0])
    pltpu.sync_copy(wT.at[:, pl.ds(extra * VBLK, VBLK)], wb.at[0])
    compute_panel(0)
    pltpu.sync_copy(pb.at[0], p_out.at[pl.ds(extra * PSZ, PSZ)])

    @pl.when(wid == NW - 1)
    def _():
        last = NPIPE * NW + NW
        pltpu.sync_copy(embT.at[:, pl.ds(last * VBLK, VBLK)], eb.at[1])
        pltpu.sync_copy(wT.at[:, pl.ds(last * VBLK, VBLK)], wb.at[1])
        compute_panel(1)
        pltpu.sync_copy(pb.at[1], p_out.at[pl.ds(last * PSZ, PSZ)])

        pltpu.sync_copy(tail_p, tb)
        for r in range(V_TAIL * EMBED_DIM // 128):
            for h in range(8):
                pb[0, pl.ds(r * 128 + h * 16, 16)] = tb[r, pl.ds(h * 16, 16)]
        pltpu.sync_copy(
            pb.at[0, pl.ds(0, V_TAIL * EMBED_DIM)],
            p_out.at[pl.ds((last + 1) * PSZ, V_TAIL * EMBED_DIM)])


def _phase2(idxT, p_in, out, ib, tid, offb, gb, ob,
            sg0, sg1, so0, so1):
    wid = lax.axis_index("s") * 2 + lax.axis_index("c")
    sg = (sg0, sg1)
    so = (so0, so1)
    bc_per_f = BATCH // BBLK

    def unit_fb(k):
        u = wid + k * NW
        return u // bc_per_f, lax.rem(u, bc_per_f) * BBLK

    def prep(k, par):
        f, b0 = unit_fb(k)
        pltpu.sync_copy(idxT.at[pl.ds(f, 1), pl.ds(b0, BBLK)], ib.at[par])
        for h in range(BBLK // 16):
            s16 = pl.ds(h * 16, 16)
            iv = ib[par, 0, s16]
            tid[par, h // 8, pl.ds((h % 8) * 16, 16)] = (
                lax.shift_right_logical(iv, 2))
            offb[par, s16] = lax.rem(iv, PACK) * EMBED_DIM
        for j in range(BBLK // 128):
            pltpu.async_copy(
                p_in.at[tid.at[par, j]],
                gb.at[par, pl.ds(j * 128, 128)], sg[par])

    def wait_gathers(par):
        for j in range(BBLK // 128):
            pltpu.make_async_copy(
                p_in.at[tid.at[par, j]],
                gb.at[par, pl.ds(j * 128, 128)], sg[par]).wait()

    for par in range(2):
        prep(par, par)

    def unit_body(k2, carry):
        for par in range(2):
            k = k2 * 2 + par
            f, b0 = unit_fb(k)
            wait_gathers(par)

            @pl.when(k2 >= 1)
            def _():
                pltpu.make_async_copy(
                    ob.at[par], out.at[0, :, pl.ds(0, BBLK)], so[par]).wait()

            def g_body(g, c2):
                offv = offb[par, pl.ds(g * 16, 16)]
                offs = [offv[kk] for kk in range(16)]
                for ch in range(2):
                    rows = [
                        gb[par, g * 16 + kk, pl.ds(offs[kk] + ch * 16, 16)]
                        for kk in range(16)
                    ]
                    t = _xpose16(rows)
                    for i in range(16):
                        ob[par, ch * 16 + i, pl.ds(g * 16, 16)] = t[i]
                return c2

            lax.fori_loop(0, BBLK // 16, g_body, 0, unroll=2)
            pltpu.async_copy(
                ob.at[par], out.at[f, :, pl.ds(b0, BBLK)], so[par])

            @pl.when(k2 < UPW // 2 - 1)
            def _():
                prep(k + 2, par)

        return carry

    lax.fori_loop(0, UPW // 2, unit_body, 0)
    for par in range(2):
        pltpu.make_async_copy(
            ob.at[par], out.at[0, :, pl.ds(0, BBLK)], so[par]).wait()


_MESH = plsc.VectorSubcoreMesh(core_axis_name="c", subcore_axis_name="s")

_p1 = functools.partial(
    pl.kernel,
    out_type=jax.ShapeDtypeStruct((P_LEN,), jnp.float32),
    mesh=_MESH,
    scratch_types=[
        pltpu.VMEM((2, EMBED_DIM, VBLK), jnp.float32),
        pltpu.VMEM((2, EMBED_DIM, VBLK), jnp.float32),
        pltpu.VMEM((2, PSZ), jnp.float32),
        pltpu.VMEM((V_TAIL * EMBED_DIM // 128, 128), jnp.float32),
        pltpu.SemaphoreType.DMA,
        pltpu.SemaphoreType.DMA,
        pltpu.SemaphoreType.DMA,
        pltpu.SemaphoreType.DMA,
    ],
)(_phase1)

_p2 = functools.partial(
    pl.kernel,
    out_type=jax.ShapeDtypeStruct((N_FIELDS, EMBED_DIM, BATCH), jnp.float32),
    mesh=_MESH,
    scratch_types=[
        pltpu.VMEM((2, 1, BBLK), jnp.int32),
        pltpu.VMEM((2, BBLK // 128, 128), jnp.int32),
        pltpu.VMEM((2, BBLK), jnp.int32),
        pltpu.VMEM((2, BBLK, 128), jnp.float32),
        pltpu.VMEM((2, EMBED_DIM, BBLK), jnp.float32),
        pltpu.SemaphoreType.DMA,
        pltpu.SemaphoreType.DMA,
        pltpu.SemaphoreType.DMA,
        pltpu.SemaphoreType.DMA,
    ],
)(_phase2)


@jax.jit
def kernel(inputs, embeddings, w):
    embT = embeddings.T
    wT = w.T
    tail_v0 = (VOCAB // VBLK) * VBLK
    tail_p = (embeddings[tail_v0:] * w[tail_v0:]).reshape(
        V_TAIL * EMBED_DIM // 128, 128)
    idxT = inputs.astype(jnp.int32).T
    p = _p1(embT, wT, tail_p)
    p2d = p.reshape(P_LEN // 128, 128)
    outT = _p2(idxT, p2d)
    return outT.transpose(2, 0, 1)
